# baseline (speedup 1.0000x reference)
import functools

import jax
import jax.numpy as jnp
from jax import lax
from jax.experimental import pallas as pl
from jax.experimental.pallas import tpu as pltpu
from jax.experimental.pallas import tpu_sc as plsc

NUM_EMBEDDINGS = 8192
LATENT_DIM = 256
N_TOKENS = 8192

BT = 256
BC = 1024
N_CHUNKS = NUM_EMBEDDINGS // BC


def _wn_body(w_ref, wn_ref):
    w = w_ref[...]
    wn = jnp.sum(w * w, axis=1)
    wn_ref[...] = wn[None, :]


def _wn(embedding_weight):
    return pl.pallas_call(
        _wn_body,
        grid=(NUM_EMBEDDINGS // BC,),
        in_specs=[pl.BlockSpec((BC, LATENT_DIM), lambda c: (c, 0))],
        out_specs=pl.BlockSpec((1, BC), lambda c: (0, c)),
        out_shape=jax.ShapeDtypeStruct((1, NUM_EMBEDDINGS), jnp.float32),
    )(embedding_weight)


def _argmin_body(x_ref, w_ref, wn_ref, idx_ref):
    x = x_ref[...]
    xn = jnp.sum(x * x, axis=1, keepdims=True)
    x2 = -2.0 * x
    run_min = jnp.full((BT, 1), jnp.inf, dtype=jnp.float32)
    run_base = jnp.zeros((BT, 1), dtype=jnp.int32)
    s_win = jnp.zeros((BT, BC), dtype=jnp.float32)
    for c in range(N_CHUNKS):
        w = w_ref[pl.ds(c * BC, BC), :]
        wn = wn_ref[:, pl.ds(c * BC, BC)]
        mm = jax.lax.dot_general(
            x2, w, (((1,), (1,)), ((), ())),
            preferred_element_type=jnp.float32)
        s = (xn + wn) + mm
        bmin = jnp.min(s, axis=1, keepdims=True)
        upd = bmin < run_min
        run_min = jnp.where(upd, bmin, run_min)
        run_base = jnp.where(upd, c * BC, run_base)
        s_win = jnp.where(upd, s, s_win)
    lane = jax.lax.broadcasted_iota(
        jnp.int32, (BT, BC), 1).astype(jnp.float32)
    cand = (s_win - run_min) * 1e30 + lane
    barg = jnp.min(cand, axis=1, keepdims=True).astype(jnp.int32)
    idx_ref[...] = barg + run_base


def _argmin_indices(x, w, wn):
    return pl.pallas_call(
        _argmin_body,
        grid=(N_TOKENS // BT,),
        in_specs=[
            pl.BlockSpec((BT, LATENT_DIM), lambda t: (t, 0)),
            pl.BlockSpec((NUM_EMBEDDINGS, LATENT_DIM), lambda t: (0, 0)),
            pl.BlockSpec((1, NUM_EMBEDDINGS), lambda t: (0, 0)),
        ],
        out_specs=pl.BlockSpec((BT, 1), lambda t: (t, 0)),
        out_shape=jax.ShapeDtypeStruct((N_TOKENS, 1), jnp.int32),
    )(x, w, wn)


_SC_ROWS_PER_WORKER = N_TOKENS // 32
_SC_CHUNK = 128
_SC_NCHUNK = _SC_ROWS_PER_WORKER // _SC_CHUNK


def _sc_gather(table, idx):
    mesh = plsc.VectorSubcoreMesh(core_axis_name="c", subcore_axis_name="s")

    @functools.partial(
        pl.kernel, mesh=mesh,
        out_type=jax.ShapeDtypeStruct((N_TOKENS, LATENT_DIM), jnp.float32),
        scratch_types=[
            pltpu.VMEM((_SC_NCHUNK, _SC_CHUNK), jnp.int32),
            pltpu.VMEM((_SC_NCHUNK, _SC_CHUNK, LATENT_DIM), jnp.float32),
            pltpu.SemaphoreType.DMA,
        ],
    )
    def k(table_hbm, idx_hbm, out_hbm, idx_v, rows_v, sem):
        wid = lax.axis_index("s") * 2 + lax.axis_index("c")
        base = wid * _SC_ROWS_PER_WORKER
        for j in range(_SC_NCHUNK):
            pltpu.sync_copy(idx_hbm.at[pl.ds(base + j * _SC_CHUNK, _SC_CHUNK)],
                            idx_v.at[j])
        cps = [pltpu.async_copy(table_hbm.at[idx_v.at[j]], rows_v.at[j], sem)
               for j in range(_SC_NCHUNK)]
        for cp in cps:
            cp.wait()
        for j in range(_SC_NCHUNK):
            pltpu.sync_copy(
                rows_v.at[j],
                out_hbm.at[pl.ds(base + j * _SC_CHUNK, _SC_CHUNK)])

    return k(table, idx)


def kernel(x, embedding_weight):
    wn = _wn(embedding_weight)
    idx = _argmin_indices(x, embedding_weight, wn)
    return _sc_gather(embedding_weight, idx[:, 0])


# speedup vs baseline: 1.0325x; 1.0325x over previous
import functools

import jax
import jax.numpy as jnp
from jax import lax
from jax.experimental import pallas as pl
from jax.experimental.pallas import tpu as pltpu
from jax.experimental.pallas import tpu_sc as plsc

NUM_EMBEDDINGS = 8192
LATENT_DIM = 256
N_TOKENS = 8192

BT = 1024
BC = 1024
N_CHUNKS = NUM_EMBEDDINGS // BC


def _wn_body(w_ref, wn_ref):
    w = w_ref[...]
    wn = jnp.sum(w * w, axis=1)
    wn_ref[...] = wn[None, :]


def _wn(embedding_weight):
    return pl.pallas_call(
        _wn_body,
        grid=(NUM_EMBEDDINGS // BC,),
        in_specs=[pl.BlockSpec((BC, LATENT_DIM), lambda c: (c, 0))],
        out_specs=pl.BlockSpec((1, BC), lambda c: (0, c)),
        out_shape=jax.ShapeDtypeStruct((1, NUM_EMBEDDINGS), jnp.float32),
    )(embedding_weight)


def _argmin_body(x_ref, w_ref, wn_ref, idx_ref):
    x = x_ref[...]
    xn = jnp.sum(x * x, axis=1, keepdims=True)
    x2 = -2.0 * x
    run_min = jnp.full((BT, 1), jnp.inf, dtype=jnp.float32)
    run_base = jnp.zeros((BT, 1), dtype=jnp.int32)
    s_win = jnp.zeros((BT, BC), dtype=jnp.float32)
    for c in range(N_CHUNKS):
        w = w_ref[pl.ds(c * BC, BC), :]
        wn = wn_ref[:, pl.ds(c * BC, BC)]
        mm = jax.lax.dot_general(
            x2, w, (((1,), (1,)), ((), ())),
            preferred_element_type=jnp.float32)
        s = (xn + wn) + mm
        bmin = jnp.min(s, axis=1, keepdims=True)
        upd = bmin < run_min
        run_min = jnp.where(upd, bmin, run_min)
        run_base = jnp.where(upd, c * BC, run_base)
        s_win = jnp.where(upd, s, s_win)
    lane = jax.lax.broadcasted_iota(
        jnp.int32, (BT, BC), 1).astype(jnp.float32)
    cand = (s_win - run_min) * 1e30 + lane
    barg = jnp.min(cand, axis=1, keepdims=True).astype(jnp.int32)
    idx_ref[...] = barg + run_base


def _argmin_indices(x, w, wn):
    return pl.pallas_call(
        _argmin_body,
        grid=(N_TOKENS // BT,),
        in_specs=[
            pl.BlockSpec((BT, LATENT_DIM), lambda t: (t, 0)),
            pl.BlockSpec((NUM_EMBEDDINGS, LATENT_DIM), lambda t: (0, 0)),
            pl.BlockSpec((1, NUM_EMBEDDINGS), lambda t: (0, 0)),
        ],
        out_specs=pl.BlockSpec((BT, 1), lambda t: (t, 0)),
        out_shape=jax.ShapeDtypeStruct((N_TOKENS, 1), jnp.int32),
    )(x, w, wn)


_SC_ROWS_PER_WORKER = N_TOKENS // 32
_SC_CHUNK = 128
_SC_NCHUNK = _SC_ROWS_PER_WORKER // _SC_CHUNK


def _sc_gather(table, idx):
    mesh = plsc.VectorSubcoreMesh(core_axis_name="c", subcore_axis_name="s")

    @functools.partial(
        pl.kernel, mesh=mesh,
        out_type=jax.ShapeDtypeStruct((N_TOKENS, LATENT_DIM), jnp.float32),
        scratch_types=[
            pltpu.VMEM((_SC_NCHUNK, _SC_CHUNK), jnp.int32),
            pltpu.VMEM((_SC_NCHUNK, _SC_CHUNK, LATENT_DIM), jnp.float32),
            pltpu.SemaphoreType.DMA,
        ],
    )
    def k(table_hbm, idx_hbm, out_hbm, idx_v, rows_v, sem):
        wid = lax.axis_index("s") * 2 + lax.axis_index("c")
        base = wid * _SC_ROWS_PER_WORKER
        for j in range(_SC_NCHUNK):
            pltpu.sync_copy(idx_hbm.at[pl.ds(base + j * _SC_CHUNK, _SC_CHUNK)],
                            idx_v.at[j])
        cps = [pltpu.async_copy(table_hbm.at[idx_v.at[j]], rows_v.at[j], sem)
               for j in range(_SC_NCHUNK)]
        for cp in cps:
            cp.wait()
        for j in range(_SC_NCHUNK):
            pltpu.sync_copy(
                rows_v.at[j],
                out_hbm.at[pl.ds(base + j * _SC_CHUNK, _SC_CHUNK)])

    return k(table, idx)


def kernel(x, embedding_weight):
    wn = _wn(embedding_weight)
    idx = _argmin_indices(x, embedding_weight, wn)
    return _sc_gather(embedding_weight, idx[:, 0])


# speedup vs baseline: 1.0870x; 1.0527x over previous
import functools

import jax
import jax.numpy as jnp
from jax import lax
from jax.experimental import pallas as pl
from jax.experimental.pallas import tpu as pltpu
from jax.experimental.pallas import tpu_sc as plsc

NUM_EMBEDDINGS = 8192
LATENT_DIM = 256
N_TOKENS = 8192

BT = 512
BC = 1024
N_CHUNKS = NUM_EMBEDDINGS // BC


def _argmin_body(x_ref, w_ref, idx_ref, wn_ref):
    @pl.when(pl.program_id(0) == 0)
    def _():
        for c in range(N_CHUNKS):
            wc = w_ref[pl.ds(c * BC, BC), :]
            wn_ref[:, pl.ds(c * BC, BC)] = jnp.sum(wc * wc, axis=1)[None, :]

    x = x_ref[...]
    xn = jnp.sum(x * x, axis=1, keepdims=True)
    x2 = -2.0 * x
    run_min = jnp.full((BT, 1), jnp.inf, dtype=jnp.float32)
    run_base = jnp.zeros((BT, 1), dtype=jnp.int32)
    s_win = jnp.zeros((BT, BC), dtype=jnp.float32)
    for c in range(N_CHUNKS):
        w = w_ref[pl.ds(c * BC, BC), :]
        wn = wn_ref[:, pl.ds(c * BC, BC)]
        mm = jax.lax.dot_general(
            x2, w, (((1,), (1,)), ((), ())),
            preferred_element_type=jnp.float32)
        s = (xn + wn) + mm
        bmin = jnp.min(s, axis=1, keepdims=True)
        upd = bmin < run_min
        run_min = jnp.where(upd, bmin, run_min)
        run_base = jnp.where(upd, c * BC, run_base)
        s_win = jnp.where(upd, s, s_win)
    lane = jax.lax.broadcasted_iota(
        jnp.int32, (BT, BC), 1).astype(jnp.float32)
    cand = (s_win - run_min) * 1e30 + lane
    barg = jnp.min(cand, axis=1, keepdims=True).astype(jnp.int32)
    idx_ref[...] = barg + run_base


def _argmin_indices(x, w):
    return pl.pallas_call(
        _argmin_body,
        grid=(N_TOKENS // BT,),
        in_specs=[
            pl.BlockSpec((BT, LATENT_DIM), lambda t: (t, 0)),
            pl.BlockSpec((NUM_EMBEDDINGS, LATENT_DIM), lambda t: (0, 0)),
        ],
        out_specs=pl.BlockSpec((BT, 1), lambda t: (t, 0)),
        out_shape=jax.ShapeDtypeStruct((N_TOKENS, 1), jnp.int32),
        scratch_shapes=[pltpu.VMEM((1, NUM_EMBEDDINGS), jnp.float32)],
    )(x, w)


_SC_ROWS_PER_WORKER = N_TOKENS // 32
_SC_CHUNK = 128
_SC_NCHUNK = _SC_ROWS_PER_WORKER // _SC_CHUNK


def _sc_gather(table, idx):
    mesh = plsc.VectorSubcoreMesh(core_axis_name="c", subcore_axis_name="s")

    @functools.partial(
        pl.kernel, mesh=mesh,
        out_type=jax.ShapeDtypeStruct((N_TOKENS, LATENT_DIM), jnp.float32),
        scratch_types=[
            pltpu.VMEM((_SC_NCHUNK, _SC_CHUNK), jnp.int32),
            pltpu.VMEM((_SC_NCHUNK, _SC_CHUNK, LATENT_DIM), jnp.float32),
            pltpu.SemaphoreType.DMA,
        ],
    )
    def k(table_hbm, idx_hbm, out_hbm, idx_v, rows_v, sem):
        wid = lax.axis_index("s") * 2 + lax.axis_index("c")
        base = wid * _SC_ROWS_PER_WORKER
        for j in range(_SC_NCHUNK):
            pltpu.sync_copy(idx_hbm.at[pl.ds(base + j * _SC_CHUNK, _SC_CHUNK)],
                            idx_v.at[j])
        cps = [pltpu.async_copy(table_hbm.at[idx_v.at[j]], rows_v.at[j], sem)
               for j in range(_SC_NCHUNK)]
        for cp in cps:
            cp.wait()
        for j in range(_SC_NCHUNK):
            pltpu.sync_copy(
                rows_v.at[j],
                out_hbm.at[pl.ds(base + j * _SC_CHUNK, _SC_CHUNK)])

    return k(table, idx)


def kernel(x, embedding_weight):
    idx = _argmin_indices(x, embedding_weight)
    return _sc_gather(embedding_weight, idx[:, 0])


# speedup vs baseline: 1.1036x; 1.0153x over previous
import functools

import jax
import jax.numpy as jnp
from jax import lax
from jax.experimental import pallas as pl
from jax.experimental.pallas import tpu as pltpu
from jax.experimental.pallas import tpu_sc as plsc

NUM_EMBEDDINGS = 8192
LATENT_DIM = 256
N_TOKENS = 8192

BT = 512
BC = 512
N_CHUNKS = NUM_EMBEDDINGS // BC


def _argmin_body(x_ref, w_ref, idx_ref, wn_ref):
    @pl.when(pl.program_id(0) == 0)
    def _():
        for c in range(N_CHUNKS):
            wc = w_ref[pl.ds(c * BC, BC), :]
            wn_ref[:, pl.ds(c * BC, BC)] = jnp.sum(wc * wc, axis=1)[None, :]

    x = x_ref[...]
    xn = jnp.sum(x * x, axis=1, keepdims=True)
    x2 = -2.0 * x
    run_min = jnp.full((BT, 1), jnp.inf, dtype=jnp.float32)
    run_base = jnp.zeros((BT, 1), dtype=jnp.int32)
    s_win = jnp.zeros((BT, BC), dtype=jnp.float32)
    for c in range(N_CHUNKS):
        w = w_ref[pl.ds(c * BC, BC), :]
        wn = wn_ref[:, pl.ds(c * BC, BC)]
        mm = jax.lax.dot_general(
            x2, w, (((1,), (1,)), ((), ())),
            preferred_element_type=jnp.float32)
        s = (xn + wn) + mm
        bmin = jnp.min(s, axis=1, keepdims=True)
        upd = bmin < run_min
        run_min = jnp.where(upd, bmin, run_min)
        run_base = jnp.where(upd, c * BC, run_base)
        s_win = jnp.where(upd, s, s_win)
    lane = jax.lax.broadcasted_iota(
        jnp.int32, (BT, BC), 1).astype(jnp.float32)
    cand = (s_win - run_min) * 1e30 + lane
    barg = jnp.min(cand, axis=1, keepdims=True).astype(jnp.int32)
    idx_ref[...] = barg + run_base


def _argmin_indices(x, w):
    return pl.pallas_call(
        _argmin_body,
        grid=(N_TOKENS // BT,),
        in_specs=[
            pl.BlockSpec((BT, LATENT_DIM), lambda t: (t, 0)),
            pl.BlockSpec((NUM_EMBEDDINGS, LATENT_DIM), lambda t: (0, 0)),
        ],
        out_specs=pl.BlockSpec((BT, 1), lambda t: (t, 0)),
        out_shape=jax.ShapeDtypeStruct((N_TOKENS, 1), jnp.int32),
        scratch_shapes=[pltpu.VMEM((1, NUM_EMBEDDINGS), jnp.float32)],
    )(x, w)


_SC_ROWS_PER_WORKER = N_TOKENS // 32
_SC_CHUNK = 128
_SC_NCHUNK = _SC_ROWS_PER_WORKER // _SC_CHUNK


def _sc_gather(table, idx):
    mesh = plsc.VectorSubcoreMesh(core_axis_name="c", subcore_axis_name="s")

    @functools.partial(
        pl.kernel, mesh=mesh,
        out_type=jax.ShapeDtypeStruct((N_TOKENS, LATENT_DIM), jnp.float32),
        scratch_types=[
            pltpu.VMEM((_SC_NCHUNK, _SC_CHUNK), jnp.int32),
            pltpu.VMEM((_SC_NCHUNK, _SC_CHUNK, LATENT_DIM), jnp.float32),
            pltpu.SemaphoreType.DMA,
        ],
    )
    def k(table_hbm, idx_hbm, out_hbm, idx_v, rows_v, sem):
        wid = lax.axis_index("s") * 2 + lax.axis_index("c")
        base = wid * _SC_ROWS_PER_WORKER
        for j in range(_SC_NCHUNK):
            pltpu.sync_copy(idx_hbm.at[pl.ds(base + j * _SC_CHUNK, _SC_CHUNK)],
                            idx_v.at[j])
        cps = [pltpu.async_copy(table_hbm.at[idx_v.at[j]], rows_v.at[j], sem)
               for j in range(_SC_NCHUNK)]
        for cp in cps:
            cp.wait()
        for j in range(_SC_NCHUNK):
            pltpu.sync_copy(
                rows_v.at[j],
                out_hbm.at[pl.ds(base + j * _SC_CHUNK, _SC_CHUNK)])

    return k(table, idx)


def kernel(x, embedding_weight):
    idx = _argmin_indices(x, embedding_weight)
    return _sc_gather(embedding_weight, idx[:, 0])
